# baseline (speedup 1.0000x reference)
import functools

import jax
import jax.numpy as jnp
from jax import lax
from jax.experimental import pallas as pl
from jax.experimental.pallas import tpu as pltpu
from jax.experimental.pallas import tpu_sc as plsc

N = 10000
D = 128
E = 320000
NS = 16
EPT = E // NS
K = 100
STEPS = EPT // K
CH = 40
NCH = STEPS // CH
CW = 16
DW = D + CW


def _sc_body(xu, xi, eui, eiu, zf,
             fus_i, fus_u,
             acc, src_v, dst_v, rows_a, rows_b, semA, semB):
  c = lax.axis_index("c")
  s = lax.axis_index("s")

  @pl.when(s < NS - 1)
  def _():
    pltpu.sync_copy(zf.at[pl.ds(s * 640, 640)], acc.at[pl.ds(s * 640, 640)])

  @pl.when(s == NS - 1)
  def _():
    pltpu.sync_copy(zf.at[pl.ds(9600, 400)], acc.at[pl.ds(9600, 400)])

  plsc.subcore_barrier()

  def run_type(e_hbm, x_src):
    @pl.loop(0, NCH)
    def _(ch):
      pltpu.sync_copy(e_hbm.at[0, s, pl.ds(ch * CH, CH)], src_v)
      pltpu.sync_copy(e_hbm.at[1, s, pl.ds(ch * CH, CH)], dst_v)
      pltpu.async_copy(x_src.at[src_v.at[0]], rows_a, semA)

      @pl.loop(0, CH // 2)
      def _(h):
        j = 2 * h
        pltpu.async_copy(x_src.at[src_v.at[j + 1]], rows_b, semB)
        pltpu.make_async_copy(x_src.at[src_v.at[j]], rows_a, semA).wait()
        pltpu.sync_copy(rows_a, acc.at[dst_v.at[j]], add=True)

        @pl.when(j + 2 < CH)
        def _():
          pltpu.async_copy(x_src.at[src_v.at[j + 2]], rows_a, semA)

        pltpu.make_async_copy(x_src.at[src_v.at[j + 1]], rows_b, semB).wait()
        pltpu.sync_copy(rows_b, acc.at[dst_v.at[j + 1]], add=True)

  @pl.when(c == 0)
  def _():
    run_type(eui, xu)

  @pl.when(c == 1)
  def _():
    run_type(eiu, xi)

  plsc.subcore_barrier()

  def write_out(fus_o):
    @pl.when(s < NS - 1)
    def _():
      pltpu.sync_copy(acc.at[pl.ds(s * 640, 640)], fus_o.at[pl.ds(s * 640, 640)])

    @pl.when(s == NS - 1)
    def _():
      pltpu.sync_copy(acc.at[pl.ds(9600, 400)], fus_o.at[pl.ds(9600, 400)])

  @pl.when(c == 0)
  def _():
    write_out(fus_i)

  @pl.when(c == 1)
  def _():
    write_out(fus_u)


_sc_segment_sums = pl.kernel(
    _sc_body,
    out_type=[
        jax.ShapeDtypeStruct((N, DW), jnp.float32),
        jax.ShapeDtypeStruct((N, DW), jnp.float32),
    ],
    mesh=plsc.VectorSubcoreMesh(core_axis_name="c", subcore_axis_name="s"),
    scratch_types=[
        pltpu.VMEM_SHARED((N, DW), jnp.float32),
        pltpu.VMEM((CH, K), jnp.int32),
        pltpu.VMEM((CH, K), jnp.int32),
        pltpu.VMEM((K, DW), jnp.float32),
        pltpu.VMEM((K, DW), jnp.float32),
        pltpu.SemaphoreType.DMA,
        pltpu.SemaphoreType.DMA,
    ],
    compiler_params=pltpu.CompilerParams(use_tc_tiling_on_sc=False),
)


def _matmul_t(a, w):
  return lax.dot_general(a, w, (((1,), (1,)), ((), ())),
                         preferred_element_type=jnp.float32,
                         precision=lax.Precision.HIGHEST)


def _tc_pre_body(xu, wr_iu, bl_iu, xi, wr_ui, bl_ui, pre_u, pre_i):
  pre_u[...] = _matmul_t(xu[...], wr_iu[...]) + bl_iu[...]
  pre_i[...] = _matmul_t(xi[...], wr_ui[...]) + bl_ui[...]


def _tc_body(fus_u, pre_u, wl_iu, g_u, b_u,
             fus_i, pre_i, wl_ui, g_i, b_i,
             out_u, out_i):
  def post(fus, pre, wl, g, b):
    mean = fus[:, 0:D] / jnp.maximum(fus[:, D:D + 1], 1.0)
    y = _matmul_t(mean, wl) + pre
    mu = jnp.mean(y, axis=-1, keepdims=True)
    var = jnp.mean((y - mu) ** 2, axis=-1, keepdims=True)
    yn = (y - mu) * lax.rsqrt(var + 1e-5) * g + b
    return yn * 0.5 * (1.0 + lax.erf(yn * 0.7071067811865476))

  out_u[...] = post(fus_u[...], pre_u[...], wl_iu[...], g_u[...], b_u[...])
  out_i[...] = post(fus_i[...], pre_i[...], wl_ui[...], g_i[...], b_i[...])


_TC_BLOCK = 2000
_row_spec = pl.BlockSpec((_TC_BLOCK, D), lambda i: (i, 0))
_fus_spec = pl.BlockSpec((_TC_BLOCK, DW), lambda i: (i, 0))
_w_spec = pl.BlockSpec((D, D), lambda i: (0, 0))
_v_spec = pl.BlockSpec((D,), lambda i: (0,))
_row_out = [jax.ShapeDtypeStruct((N, D), jnp.float32),
            jax.ShapeDtypeStruct((N, D), jnp.float32)]


def _tc_pre(*args):
  per_type = [_row_spec, _w_spec, _v_spec]
  return pl.pallas_call(
      _tc_pre_body,
      grid=(N // _TC_BLOCK,),
      in_specs=per_type + per_type,
      out_specs=[_row_spec, _row_spec],
      out_shape=_row_out,
  )(*args)


def _tc_call(*args):
  per_type = [_fus_spec, _row_spec, _w_spec, _v_spec, _v_spec]
  return pl.pallas_call(
      _tc_body,
      grid=(N // _TC_BLOCK,),
      in_specs=per_type + per_type,
      out_specs=[_row_spec, _row_spec],
      out_shape=_row_out,
  )(*args)


def kernel(x_user, x_item, edge_ui, edge_iu, Wl_ui, bl_ui, Wr_ui,
           Wl_iu, bl_iu, Wr_iu, g_user, b_user, g_item, b_item):
  eui = edge_ui.reshape(2, NS, STEPS, K)
  eiu = edge_iu.reshape(2, NS, STEPS, K)
  zf = jnp.zeros((N, DW), jnp.float32)
  ones_col = jnp.ones((N, CW), jnp.float32)
  xau = jnp.concatenate([x_user, ones_col], axis=1)
  xai = jnp.concatenate([x_item, ones_col], axis=1)
  fus_i, fus_u = _sc_segment_sums(xau, xai, eui, eiu, zf)
  pre_u, pre_i = _tc_pre(x_user, Wr_iu, bl_iu, x_item, Wr_ui, bl_ui)
  out_u, out_i = _tc_call(
      fus_u, pre_u, Wl_iu, g_user, b_user,
      fus_i, pre_i, Wl_ui, g_item, b_item)
  return (out_u, out_i)


# speedup vs baseline: 1.1562x; 1.1562x over previous
import functools

import jax
import jax.numpy as jnp
from jax import lax
from jax.experimental import pallas as pl
from jax.experimental.pallas import tpu as pltpu
from jax.experimental.pallas import tpu_sc as plsc

N = 10000
D = 128
E = 320000
NS = 16
EPT = E // NS
K = 100
STEPS = EPT // K
CH = 40
NCH = STEPS // CH
CW = 16


def _sc_body(xu, xi, eui, eiu, zf, zc,
             sum_i, cnt_i, sum_u, cnt_u,
             acc, cacc, src_v, dst_v, rows_a, rows_b, ones_v, semA, semB):
  c = lax.axis_index("c")
  s = lax.axis_index("s")

  @pl.loop(0, K)
  def _(j):
    ones_v[j, :] = jnp.ones((16,), jnp.float32)

  @pl.when(s < NS - 1)
  def _():
    pltpu.sync_copy(zf.at[pl.ds(s * 640, 640)], acc.at[pl.ds(s * 640, 640)])
    pltpu.sync_copy(zc.at[pl.ds(s * 640, 640)], cacc.at[pl.ds(s * 640, 640)])

  @pl.when(s == NS - 1)
  def _():
    pltpu.sync_copy(zf.at[pl.ds(9600, 400)], acc.at[pl.ds(9600, 400)])
    pltpu.sync_copy(zc.at[pl.ds(9600, 400)], cacc.at[pl.ds(9600, 400)])

  plsc.subcore_barrier()

  def run_type(e_hbm, x_src):
    @pl.loop(0, NCH)
    def _(ch):
      pltpu.sync_copy(e_hbm.at[0, s, pl.ds(ch * CH, CH)], src_v)
      pltpu.sync_copy(e_hbm.at[1, s, pl.ds(ch * CH, CH)], dst_v)
      pltpu.async_copy(x_src.at[src_v.at[0]], rows_a, semA)

      @pl.loop(0, CH // 2)
      def _(h):
        j = 2 * h
        pltpu.async_copy(x_src.at[src_v.at[j + 1]], rows_b, semB)
        pltpu.make_async_copy(x_src.at[src_v.at[j]], rows_a, semA).wait()
        pltpu.sync_copy(rows_a, acc.at[dst_v.at[j]], add=True)
        pltpu.sync_copy(ones_v, cacc.at[dst_v.at[j]], add=True)

        @pl.when(j + 2 < CH)
        def _():
          pltpu.async_copy(x_src.at[src_v.at[j + 2]], rows_a, semA)

        pltpu.make_async_copy(x_src.at[src_v.at[j + 1]], rows_b, semB).wait()
        pltpu.sync_copy(rows_b, acc.at[dst_v.at[j + 1]], add=True)
        pltpu.sync_copy(ones_v, cacc.at[dst_v.at[j + 1]], add=True)

  @pl.when(c == 0)
  def _():
    run_type(eui, xu)

  @pl.when(c == 1)
  def _():
    run_type(eiu, xi)

  plsc.subcore_barrier()

  def write_out(sum_o, cnt_o):
    @pl.when(s < NS - 1)
    def _():
      pltpu.sync_copy(acc.at[pl.ds(s * 640, 640)], sum_o.at[pl.ds(s * 640, 640)])
      pltpu.sync_copy(cacc.at[pl.ds(s * 640, 640)], cnt_o.at[pl.ds(s * 640, 640)])

    @pl.when(s == NS - 1)
    def _():
      pltpu.sync_copy(acc.at[pl.ds(9600, 400)], sum_o.at[pl.ds(9600, 400)])
      pltpu.sync_copy(cacc.at[pl.ds(9600, 400)], cnt_o.at[pl.ds(9600, 400)])

  @pl.when(c == 0)
  def _():
    write_out(sum_i, cnt_i)

  @pl.when(c == 1)
  def _():
    write_out(sum_u, cnt_u)


_sc_segment_sums = pl.kernel(
    _sc_body,
    out_type=[
        jax.ShapeDtypeStruct((N, D), jnp.float32),
        jax.ShapeDtypeStruct((N, CW), jnp.float32),
        jax.ShapeDtypeStruct((N, D), jnp.float32),
        jax.ShapeDtypeStruct((N, CW), jnp.float32),
    ],
    mesh=plsc.VectorSubcoreMesh(core_axis_name="c", subcore_axis_name="s"),
    scratch_types=[
        pltpu.VMEM_SHARED((N, D), jnp.float32),
        pltpu.VMEM_SHARED((N, CW), jnp.float32),
        pltpu.VMEM((CH, K), jnp.int32),
        pltpu.VMEM((CH, K), jnp.int32),
        pltpu.VMEM((K, D), jnp.float32),
        pltpu.VMEM((K, D), jnp.float32),
        pltpu.VMEM((K, CW), jnp.float32),
        pltpu.SemaphoreType.DMA,
        pltpu.SemaphoreType.DMA,
    ],
    compiler_params=pltpu.CompilerParams(use_tc_tiling_on_sc=False),
)


def _matmul_t(a, w):
  return lax.dot_general(a, w, (((1,), (1,)), ((), ())),
                         preferred_element_type=jnp.float32,
                         precision=lax.Precision.HIGHEST)


def _tc_pre_body(xu, wr_iu, bl_iu, xi, wr_ui, bl_ui, pre_u, pre_i):
  pre_u[...] = _matmul_t(xu[...], wr_iu[...]) + bl_iu[...]
  pre_i[...] = _matmul_t(xi[...], wr_ui[...]) + bl_ui[...]


def _tc_body(sum_u, cnt_u, pre_u, wl_iu, g_u, b_u,
             sum_i, cnt_i, pre_i, wl_ui, g_i, b_i,
             out_u, out_i):
  def post(summed, cnt, pre, wl, g, b):
    mean = summed / jnp.maximum(cnt[:, 0:1], 1.0)
    y = _matmul_t(mean, wl) + pre
    mu = jnp.mean(y, axis=-1, keepdims=True)
    var = jnp.mean((y - mu) ** 2, axis=-1, keepdims=True)
    yn = (y - mu) * lax.rsqrt(var + 1e-5) * g + b
    return yn * 0.5 * (1.0 + lax.erf(yn * 0.7071067811865476))

  out_u[...] = post(sum_u[...], cnt_u[...], pre_u[...],
                    wl_iu[...], g_u[...], b_u[...])
  out_i[...] = post(sum_i[...], cnt_i[...], pre_i[...],
                    wl_ui[...], g_i[...], b_i[...])


_TC_BLOCK = 2000
_row_spec = pl.BlockSpec((_TC_BLOCK, D), lambda i: (i, 0))
_cnt_spec = pl.BlockSpec((_TC_BLOCK, CW), lambda i: (i, 0))
_w_spec = pl.BlockSpec((D, D), lambda i: (0, 0))
_v_spec = pl.BlockSpec((D,), lambda i: (0,))
_row_out = [jax.ShapeDtypeStruct((N, D), jnp.float32),
            jax.ShapeDtypeStruct((N, D), jnp.float32)]


def _tc_pre(*args):
  per_type = [_row_spec, _w_spec, _v_spec]
  return pl.pallas_call(
      _tc_pre_body,
      grid=(N // _TC_BLOCK,),
      in_specs=per_type + per_type,
      out_specs=[_row_spec, _row_spec],
      out_shape=_row_out,
  )(*args)


def _tc_call(*args):
  per_type = [_row_spec, _cnt_spec, _row_spec, _w_spec, _v_spec, _v_spec]
  return pl.pallas_call(
      _tc_body,
      grid=(N // _TC_BLOCK,),
      in_specs=per_type + per_type,
      out_specs=[_row_spec, _row_spec],
      out_shape=_row_out,
  )(*args)


def kernel(x_user, x_item, edge_ui, edge_iu, Wl_ui, bl_ui, Wr_ui,
           Wl_iu, bl_iu, Wr_iu, g_user, b_user, g_item, b_item):
  eui = edge_ui.reshape(2, NS, STEPS, K)
  eiu = edge_iu.reshape(2, NS, STEPS, K)
  zf = jnp.zeros((N, D), jnp.float32)
  zc = jnp.zeros((N, CW), jnp.float32)
  sum_i, cnt_i, sum_u, cnt_u = _sc_segment_sums(
      x_user, x_item, eui, eiu, zf, zc)
  pre_u, pre_i = _tc_pre(x_user, Wr_iu, bl_iu, x_item, Wr_ui, bl_ui)
  out_u, out_i = _tc_call(
      sum_u, cnt_u, pre_u, Wl_iu, g_user, b_user,
      sum_i, cnt_i, pre_i, Wl_ui, g_item, b_item)
  return (out_u, out_i)


# speedup vs baseline: 1.2644x; 1.0936x over previous
import functools

import jax
import jax.numpy as jnp
from jax import lax
from jax.experimental import pallas as pl
from jax.experimental.pallas import tpu as pltpu
from jax.experimental.pallas import tpu_sc as plsc

N = 10000
D = 128
E = 320000
NS = 16
EPT = E // NS
K = 100
STEPS = EPT // K
CH = 40
NCH = STEPS // CH
CW = 32


def _sc_body(xu, xi, eui, eiu, zf, zc,
             sum_i, cnt_i, sum_u, cnt_u,
             acc, cacc, src_v, dst_v, rows_a, rows_b, ones_v, semA, semB):
  c = lax.axis_index("c")
  s = lax.axis_index("s")

  @pl.loop(0, K)
  def _(j):
    ones_v[j, :] = jnp.ones((32,), jnp.bfloat16)

  @pl.when(s < NS - 1)
  def _():
    pltpu.sync_copy(zf.at[pl.ds(s * 640, 640)], acc.at[pl.ds(s * 640, 640)])
    pltpu.sync_copy(zc.at[pl.ds(s * 640, 640)], cacc.at[pl.ds(s * 640, 640)])

  @pl.when(s == NS - 1)
  def _():
    pltpu.sync_copy(zf.at[pl.ds(9600, 400)], acc.at[pl.ds(9600, 400)])
    pltpu.sync_copy(zc.at[pl.ds(9600, 400)], cacc.at[pl.ds(9600, 400)])

  plsc.subcore_barrier()

  def run_type(e_hbm, x_src):
    @pl.loop(0, NCH)
    def _(ch):
      pltpu.sync_copy(e_hbm.at[0, s, pl.ds(ch * CH, CH)], src_v)
      pltpu.sync_copy(e_hbm.at[1, s, pl.ds(ch * CH, CH)], dst_v)
      pltpu.async_copy(x_src.at[src_v.at[0]], rows_a, semA)

      @pl.loop(0, CH // 2)
      def _(h):
        j = 2 * h
        pltpu.async_copy(x_src.at[src_v.at[j + 1]], rows_b, semB)
        pltpu.make_async_copy(x_src.at[src_v.at[j]], rows_a, semA).wait()
        pltpu.sync_copy(rows_a, acc.at[dst_v.at[j]], add=True)
        pltpu.sync_copy(ones_v, cacc.at[dst_v.at[j]], add=True)

        @pl.when(j + 2 < CH)
        def _():
          pltpu.async_copy(x_src.at[src_v.at[j + 2]], rows_a, semA)

        pltpu.make_async_copy(x_src.at[src_v.at[j + 1]], rows_b, semB).wait()
        pltpu.sync_copy(rows_b, acc.at[dst_v.at[j + 1]], add=True)
        pltpu.sync_copy(ones_v, cacc.at[dst_v.at[j + 1]], add=True)

  @pl.when(c == 0)
  def _():
    run_type(eui, xu)

  @pl.when(c == 1)
  def _():
    run_type(eiu, xi)

  plsc.subcore_barrier()

  def write_out(sum_o, cnt_o):
    @pl.when(s < NS - 1)
    def _():
      pltpu.sync_copy(acc.at[pl.ds(s * 640, 640)], sum_o.at[pl.ds(s * 640, 640)])
      pltpu.sync_copy(cacc.at[pl.ds(s * 640, 640)], cnt_o.at[pl.ds(s * 640, 640)])

    @pl.when(s == NS - 1)
    def _():
      pltpu.sync_copy(acc.at[pl.ds(9600, 400)], sum_o.at[pl.ds(9600, 400)])
      pltpu.sync_copy(cacc.at[pl.ds(9600, 400)], cnt_o.at[pl.ds(9600, 400)])

  @pl.when(c == 0)
  def _():
    write_out(sum_i, cnt_i)

  @pl.when(c == 1)
  def _():
    write_out(sum_u, cnt_u)


_sc_segment_sums = pl.kernel(
    _sc_body,
    out_type=[
        jax.ShapeDtypeStruct((N, D), jnp.bfloat16),
        jax.ShapeDtypeStruct((N, CW), jnp.bfloat16),
        jax.ShapeDtypeStruct((N, D), jnp.bfloat16),
        jax.ShapeDtypeStruct((N, CW), jnp.bfloat16),
    ],
    mesh=plsc.VectorSubcoreMesh(core_axis_name="c", subcore_axis_name="s"),
    scratch_types=[
        pltpu.VMEM_SHARED((N, D), jnp.bfloat16),
        pltpu.VMEM_SHARED((N, CW), jnp.bfloat16),
        pltpu.VMEM((CH, K), jnp.int32),
        pltpu.VMEM((CH, K), jnp.int32),
        pltpu.VMEM((K, D), jnp.bfloat16),
        pltpu.VMEM((K, D), jnp.bfloat16),
        pltpu.VMEM((K, CW), jnp.bfloat16),
        pltpu.SemaphoreType.DMA,
        pltpu.SemaphoreType.DMA,
    ],
    compiler_params=pltpu.CompilerParams(use_tc_tiling_on_sc=False),
)


def _matmul_t(a, w):
  return lax.dot_general(a, w, (((1,), (1,)), ((), ())),
                         preferred_element_type=jnp.float32,
                         precision=lax.Precision.HIGHEST)


def _tc_pre_body(xu, wr_iu, bl_iu, xi, wr_ui, bl_ui, pre_u, pre_i):
  pre_u[...] = _matmul_t(xu[...], wr_iu[...]) + bl_iu[...]
  pre_i[...] = _matmul_t(xi[...], wr_ui[...]) + bl_ui[...]


def _tc_body(sum_u, cnt_u, pre_u, wl_iu, g_u, b_u,
             sum_i, cnt_i, pre_i, wl_ui, g_i, b_i,
             out_u, out_i):
  def post(summed, cnt, pre, wl, g, b):
    mean = (summed.astype(jnp.float32)
            / jnp.maximum(cnt[:, 0:1].astype(jnp.float32), 1.0))
    y = _matmul_t(mean, wl) + pre
    mu = jnp.mean(y, axis=-1, keepdims=True)
    var = jnp.mean((y - mu) ** 2, axis=-1, keepdims=True)
    yn = (y - mu) * lax.rsqrt(var + 1e-5) * g + b
    return yn * 0.5 * (1.0 + lax.erf(yn * 0.7071067811865476))

  out_u[...] = post(sum_u[...], cnt_u[...], pre_u[...],
                    wl_iu[...], g_u[...], b_u[...])
  out_i[...] = post(sum_i[...], cnt_i[...], pre_i[...],
                    wl_ui[...], g_i[...], b_i[...])


_TC_BLOCK = 2000
_row_spec = pl.BlockSpec((_TC_BLOCK, D), lambda i: (i, 0))
_sum_spec = pl.BlockSpec((_TC_BLOCK, D), lambda i: (i, 0))
_cnt_spec = pl.BlockSpec((_TC_BLOCK, CW), lambda i: (i, 0))
_w_spec = pl.BlockSpec((D, D), lambda i: (0, 0))
_v_spec = pl.BlockSpec((D,), lambda i: (0,))
_row_out = [jax.ShapeDtypeStruct((N, D), jnp.float32),
            jax.ShapeDtypeStruct((N, D), jnp.float32)]


def _tc_pre(*args):
  per_type = [_row_spec, _w_spec, _v_spec]
  return pl.pallas_call(
      _tc_pre_body,
      grid=(N // _TC_BLOCK,),
      in_specs=per_type + per_type,
      out_specs=[_row_spec, _row_spec],
      out_shape=_row_out,
  )(*args)


def _tc_call(*args):
  per_type = [_sum_spec, _cnt_spec, _row_spec, _w_spec, _v_spec, _v_spec]
  return pl.pallas_call(
      _tc_body,
      grid=(N // _TC_BLOCK,),
      in_specs=per_type + per_type,
      out_specs=[_row_spec, _row_spec],
      out_shape=_row_out,
  )(*args)


def kernel(x_user, x_item, edge_ui, edge_iu, Wl_ui, bl_ui, Wr_ui,
           Wl_iu, bl_iu, Wr_iu, g_user, b_user, g_item, b_item):
  eui = edge_ui.reshape(2, NS, STEPS, K)
  eiu = edge_iu.reshape(2, NS, STEPS, K)
  zf = jnp.zeros((N, D), jnp.bfloat16)
  zc = jnp.zeros((N, CW), jnp.bfloat16)
  sum_i, cnt_i, sum_u, cnt_u = _sc_segment_sums(
      x_user.astype(jnp.bfloat16), x_item.astype(jnp.bfloat16),
      eui, eiu, zf, zc)
  pre_u, pre_i = _tc_pre(x_user, Wr_iu, bl_iu, x_item, Wr_ui, bl_ui)
  out_u, out_i = _tc_call(
      sum_u, cnt_u, pre_u, Wl_iu, g_user, b_user,
      sum_i, cnt_i, pre_i, Wl_ui, g_item, b_item)
  return (out_u, out_i)


# speedup vs baseline: 1.5353x; 1.2143x over previous
import functools

import jax
import jax.numpy as jnp
from jax import lax
from jax.experimental import pallas as pl
from jax.experimental.pallas import tpu as pltpu
from jax.experimental.pallas import tpu_sc as plsc

N = 10000
D = 128
E = 320000
NS = 16
EPT = E // NS
K = 100
STEPS = EPT // K
NB = 4
CW = 32


def _sc_body(xu, xi, eui, eiu, zf, zc,
             sum_i, cnt_i, sum_u, cnt_u,
             acc, cacc, src_v, dst_v, rows, ones_v, semG, semS, semC):
  c = lax.axis_index("c")
  s = lax.axis_index("s")

  @pl.loop(0, K)
  def _(j):
    ones_v[j, :] = jnp.ones((32,), jnp.bfloat16)

  @pl.when(s < NS - 1)
  def _():
    pltpu.sync_copy(zf.at[pl.ds(s * 640, 640)], acc.at[pl.ds(s * 640, 640)])
    pltpu.sync_copy(zc.at[pl.ds(s * 640, 640)], cacc.at[pl.ds(s * 640, 640)])

  @pl.when(s == NS - 1)
  def _():
    pltpu.sync_copy(zf.at[pl.ds(9600, 400)], acc.at[pl.ds(9600, 400)])
    pltpu.sync_copy(zc.at[pl.ds(9600, 400)], cacc.at[pl.ds(9600, 400)])

  plsc.subcore_barrier()

  def run_type(e_hbm, x_src):
    pltpu.sync_copy(e_hbm.at[0, s], src_v)
    pltpu.sync_copy(e_hbm.at[1, s], dst_v)
    for q in range(NB - 1):
      pltpu.async_copy(x_src.at[src_v.at[q]], rows.at[q], semG[q])

    @pl.loop(0, STEPS // NB)
    def _(t):
      j0 = NB * t
      for q in range(NB):
        jq = j0 + q
        qn = (q + NB - 1) % NB

        @pl.when(jq >= 1)
        def _():
          pltpu.make_async_copy(rows.at[qn], acc.at[dst_v.at[0]],
                                semS[qn]).wait()
          pltpu.make_async_copy(ones_v, cacc.at[dst_v.at[0]],
                                semC[qn]).wait()

        @pl.when(jq + NB - 1 < STEPS)
        def _():
          pltpu.async_copy(x_src.at[src_v.at[jq + NB - 1]], rows.at[qn],
                           semG[qn])

        pltpu.make_async_copy(x_src.at[src_v.at[jq]], rows.at[q],
                              semG[q]).wait()
        pltpu.async_copy(rows.at[q], acc.at[dst_v.at[jq]], semS[q], add=True)
        pltpu.async_copy(ones_v, cacc.at[dst_v.at[jq]], semC[q], add=True)

    pltpu.make_async_copy(rows.at[NB - 1], acc.at[dst_v.at[0]],
                          semS[NB - 1]).wait()
    pltpu.make_async_copy(ones_v, cacc.at[dst_v.at[0]], semC[NB - 1]).wait()

  @pl.when(c == 0)
  def _():
    run_type(eui, xu)

  @pl.when(c == 1)
  def _():
    run_type(eiu, xi)

  plsc.subcore_barrier()

  def write_out(sum_o, cnt_o):
    @pl.when(s < NS - 1)
    def _():
      pltpu.sync_copy(acc.at[pl.ds(s * 640, 640)], sum_o.at[pl.ds(s * 640, 640)])
      pltpu.sync_copy(cacc.at[pl.ds(s * 640, 640)], cnt_o.at[pl.ds(s * 640, 640)])

    @pl.when(s == NS - 1)
    def _():
      pltpu.sync_copy(acc.at[pl.ds(9600, 400)], sum_o.at[pl.ds(9600, 400)])
      pltpu.sync_copy(cacc.at[pl.ds(9600, 400)], cnt_o.at[pl.ds(9600, 400)])

  @pl.when(c == 0)
  def _():
    write_out(sum_i, cnt_i)

  @pl.when(c == 1)
  def _():
    write_out(sum_u, cnt_u)


_sc_segment_sums = pl.kernel(
    _sc_body,
    out_type=[
        jax.ShapeDtypeStruct((N, D), jnp.bfloat16),
        jax.ShapeDtypeStruct((N, CW), jnp.bfloat16),
        jax.ShapeDtypeStruct((N, D), jnp.bfloat16),
        jax.ShapeDtypeStruct((N, CW), jnp.bfloat16),
    ],
    mesh=plsc.VectorSubcoreMesh(core_axis_name="c", subcore_axis_name="s"),
    scratch_types=[
        pltpu.VMEM_SHARED((N, D), jnp.bfloat16),
        pltpu.VMEM_SHARED((N, CW), jnp.bfloat16),
        pltpu.VMEM((STEPS, K), jnp.int32),
        pltpu.VMEM((STEPS, K), jnp.int32),
        pltpu.VMEM((NB, K, D), jnp.bfloat16),
        pltpu.VMEM((K, CW), jnp.bfloat16),
        [pltpu.SemaphoreType.DMA] * NB,
        [pltpu.SemaphoreType.DMA] * NB,
        [pltpu.SemaphoreType.DMA] * NB,
    ],
    compiler_params=pltpu.CompilerParams(use_tc_tiling_on_sc=False),
)


def _matmul_t(a, w):
  return lax.dot_general(a, w, (((1,), (1,)), ((), ())),
                         preferred_element_type=jnp.float32,
                         precision=lax.Precision.HIGHEST)


def _tc_pre_body(xu, wr_iu, bl_iu, xi, wr_ui, bl_ui, pre_u, pre_i):
  pre_u[...] = _matmul_t(xu[...], wr_iu[...]) + bl_iu[...]
  pre_i[...] = _matmul_t(xi[...], wr_ui[...]) + bl_ui[...]


def _tc_body(sum_u, cnt_u, pre_u, wl_iu, g_u, b_u,
             sum_i, cnt_i, pre_i, wl_ui, g_i, b_i,
             out_u, out_i):
  def post(summed, cnt, pre, wl, g, b):
    mean = (summed.astype(jnp.float32)
            / jnp.maximum(cnt[:, 0:1].astype(jnp.float32), 1.0))
    y = _matmul_t(mean, wl) + pre
    mu = jnp.mean(y, axis=-1, keepdims=True)
    var = jnp.mean((y - mu) ** 2, axis=-1, keepdims=True)
    yn = (y - mu) * lax.rsqrt(var + 1e-5) * g + b
    return yn * 0.5 * (1.0 + lax.erf(yn * 0.7071067811865476))

  out_u[...] = post(sum_u[...], cnt_u[...], pre_u[...],
                    wl_iu[...], g_u[...], b_u[...])
  out_i[...] = post(sum_i[...], cnt_i[...], pre_i[...],
                    wl_ui[...], g_i[...], b_i[...])


_TC_BLOCK = 2000
_row_spec = pl.BlockSpec((_TC_BLOCK, D), lambda i: (i, 0))
_sum_spec = pl.BlockSpec((_TC_BLOCK, D), lambda i: (i, 0))
_cnt_spec = pl.BlockSpec((_TC_BLOCK, CW), lambda i: (i, 0))
_w_spec = pl.BlockSpec((D, D), lambda i: (0, 0))
_v_spec = pl.BlockSpec((D,), lambda i: (0,))
_row_out = [jax.ShapeDtypeStruct((N, D), jnp.float32),
            jax.ShapeDtypeStruct((N, D), jnp.float32)]


def _tc_pre(*args):
  per_type = [_row_spec, _w_spec, _v_spec]
  return pl.pallas_call(
      _tc_pre_body,
      grid=(N // _TC_BLOCK,),
      in_specs=per_type + per_type,
      out_specs=[_row_spec, _row_spec],
      out_shape=_row_out,
  )(*args)


def _tc_call(*args):
  per_type = [_sum_spec, _cnt_spec, _row_spec, _w_spec, _v_spec, _v_spec]
  return pl.pallas_call(
      _tc_body,
      grid=(N // _TC_BLOCK,),
      in_specs=per_type + per_type,
      out_specs=[_row_spec, _row_spec],
      out_shape=_row_out,
  )(*args)


def kernel(x_user, x_item, edge_ui, edge_iu, Wl_ui, bl_ui, Wr_ui,
           Wl_iu, bl_iu, Wr_iu, g_user, b_user, g_item, b_item):
  eui = edge_ui.reshape(2, NS, STEPS, K)
  eiu = edge_iu.reshape(2, NS, STEPS, K)
  zf = jnp.zeros((N, D), jnp.bfloat16)
  zc = jnp.zeros((N, CW), jnp.bfloat16)
  sum_i, cnt_i, sum_u, cnt_u = _sc_segment_sums(
      x_user.astype(jnp.bfloat16), x_item.astype(jnp.bfloat16),
      eui, eiu, zf, zc)
  pre_u, pre_i = _tc_pre(x_user, Wr_iu, bl_iu, x_item, Wr_ui, bl_ui)
  out_u, out_i = _tc_call(
      sum_u, cnt_u, pre_u, Wl_iu, g_user, b_user,
      sum_i, cnt_i, pre_i, Wl_ui, g_item, b_item)
  return (out_u, out_i)


# speedup vs baseline: 1.6401x; 1.0683x over previous
import functools

import jax
import jax.numpy as jnp
from jax import lax
from jax.experimental import pallas as pl
from jax.experimental.pallas import tpu as pltpu
from jax.experimental.pallas import tpu_sc as plsc

N = 10000
D = 128
E = 320000
NS = 16
EPT = E // NS
K = 80
STEPS = EPT // K
NB = 5
CW = 32


def _sc_body(xu, xi, eui, eiu, zf, zc,
             sum_i, cnt_i, sum_u, cnt_u,
             acc, cacc, src_v, dst_v, rows, ones_v, semG, semS, semC):
  c = lax.axis_index("c")
  s = lax.axis_index("s")

  @pl.loop(0, K)
  def _(j):
    ones_v[j, :] = jnp.ones((32,), jnp.bfloat16)

  @pl.when(s < NS - 1)
  def _():
    pltpu.sync_copy(zf.at[pl.ds(s * 640, 640)], acc.at[pl.ds(s * 640, 640)])
    pltpu.sync_copy(zc.at[pl.ds(s * 640, 640)], cacc.at[pl.ds(s * 640, 640)])

  @pl.when(s == NS - 1)
  def _():
    pltpu.sync_copy(zf.at[pl.ds(9600, 400)], acc.at[pl.ds(9600, 400)])
    pltpu.sync_copy(zc.at[pl.ds(9600, 400)], cacc.at[pl.ds(9600, 400)])

  plsc.subcore_barrier()

  def run_type(e_hbm, x_src):
    pltpu.sync_copy(e_hbm.at[0, pl.ds(s * EPT, EPT)], src_v)
    pltpu.sync_copy(e_hbm.at[1, pl.ds(s * EPT, EPT)], dst_v)

    def sidx(jq):
      return src_v.at[pl.ds(jq * K, K)]

    def didx(jq):
      return dst_v.at[pl.ds(jq * K, K)]

    for q in range(NB - 1):
      pltpu.async_copy(x_src.at[sidx(q)], rows.at[q], semG[q])

    @pl.loop(0, STEPS // NB)
    def _(t):
      j0 = NB * t
      for q in range(NB):
        jq = j0 + q
        qn = (q + NB - 1) % NB

        @pl.when(jq >= 1)
        def _():
          pltpu.make_async_copy(rows.at[qn], acc.at[didx(0)], semS[qn]).wait()
          pltpu.make_async_copy(ones_v, cacc.at[didx(0)], semC[qn]).wait()

        @pl.when(jq + NB - 1 < STEPS)
        def _():
          pltpu.async_copy(x_src.at[sidx(jq + NB - 1)], rows.at[qn], semG[qn])

        pltpu.make_async_copy(x_src.at[sidx(jq)], rows.at[q], semG[q]).wait()
        pltpu.async_copy(rows.at[q], acc.at[didx(jq)], semS[q], add=True)
        pltpu.async_copy(ones_v, cacc.at[didx(jq)], semC[q], add=True)

    pltpu.make_async_copy(rows.at[NB - 1], acc.at[didx(0)],
                          semS[NB - 1]).wait()
    pltpu.make_async_copy(ones_v, cacc.at[didx(0)], semC[NB - 1]).wait()

  @pl.when(c == 0)
  def _():
    run_type(eui, xu)

  @pl.when(c == 1)
  def _():
    run_type(eiu, xi)

  plsc.subcore_barrier()

  def write_out(sum_o, cnt_o):
    @pl.when(s < NS - 1)
    def _():
      pltpu.sync_copy(acc.at[pl.ds(s * 640, 640)], sum_o.at[pl.ds(s * 640, 640)])
      pltpu.sync_copy(cacc.at[pl.ds(s * 640, 640)], cnt_o.at[pl.ds(s * 640, 640)])

    @pl.when(s == NS - 1)
    def _():
      pltpu.sync_copy(acc.at[pl.ds(9600, 400)], sum_o.at[pl.ds(9600, 400)])
      pltpu.sync_copy(cacc.at[pl.ds(9600, 400)], cnt_o.at[pl.ds(9600, 400)])

  @pl.when(c == 0)
  def _():
    write_out(sum_i, cnt_i)

  @pl.when(c == 1)
  def _():
    write_out(sum_u, cnt_u)


_sc_segment_sums = pl.kernel(
    _sc_body,
    out_type=[
        jax.ShapeDtypeStruct((N, D), jnp.bfloat16),
        jax.ShapeDtypeStruct((N, CW), jnp.bfloat16),
        jax.ShapeDtypeStruct((N, D), jnp.bfloat16),
        jax.ShapeDtypeStruct((N, CW), jnp.bfloat16),
    ],
    mesh=plsc.VectorSubcoreMesh(core_axis_name="c", subcore_axis_name="s"),
    scratch_types=[
        pltpu.VMEM_SHARED((N, D), jnp.bfloat16),
        pltpu.VMEM_SHARED((N, CW), jnp.bfloat16),
        pltpu.VMEM((EPT,), jnp.int32),
        pltpu.VMEM((EPT,), jnp.int32),
        pltpu.VMEM((NB, K, D), jnp.bfloat16),
        pltpu.VMEM((K, CW), jnp.bfloat16),
        [pltpu.SemaphoreType.DMA] * NB,
        [pltpu.SemaphoreType.DMA] * NB,
        [pltpu.SemaphoreType.DMA] * NB,
    ],
    compiler_params=pltpu.CompilerParams(use_tc_tiling_on_sc=False),
)


def _matmul_t(a, w):
  return lax.dot_general(a, w, (((1,), (1,)), ((), ())),
                         preferred_element_type=jnp.float32,
                         precision=lax.Precision.HIGHEST)


def _tc_pre_body(xu, wr_iu, bl_iu, xi, wr_ui, bl_ui, pre_u, pre_i):
  pre_u[...] = _matmul_t(xu[...], wr_iu[...]) + bl_iu[...]
  pre_i[...] = _matmul_t(xi[...], wr_ui[...]) + bl_ui[...]


def _tc_body(sum_u, cnt_u, pre_u, wl_iu, g_u, b_u,
             sum_i, cnt_i, pre_i, wl_ui, g_i, b_i,
             out_u, out_i):
  def post(summed, cnt, pre, wl, g, b):
    mean = (summed.astype(jnp.float32)
            / jnp.maximum(cnt[:, 0:1].astype(jnp.float32), 1.0))
    y = _matmul_t(mean, wl) + pre
    mu = jnp.mean(y, axis=-1, keepdims=True)
    var = jnp.mean((y - mu) ** 2, axis=-1, keepdims=True)
    yn = (y - mu) * lax.rsqrt(var + 1e-5) * g + b
    return yn * 0.5 * (1.0 + lax.erf(yn * 0.7071067811865476))

  out_u[...] = post(sum_u[...], cnt_u[...], pre_u[...],
                    wl_iu[...], g_u[...], b_u[...])
  out_i[...] = post(sum_i[...], cnt_i[...], pre_i[...],
                    wl_ui[...], g_i[...], b_i[...])


_TC_BLOCK = 2000
_row_spec = pl.BlockSpec((_TC_BLOCK, D), lambda i: (i, 0))
_sum_spec = pl.BlockSpec((_TC_BLOCK, D), lambda i: (i, 0))
_cnt_spec = pl.BlockSpec((_TC_BLOCK, CW), lambda i: (i, 0))
_w_spec = pl.BlockSpec((D, D), lambda i: (0, 0))
_v_spec = pl.BlockSpec((D,), lambda i: (0,))
_row_out = [jax.ShapeDtypeStruct((N, D), jnp.float32),
            jax.ShapeDtypeStruct((N, D), jnp.float32)]


def _tc_pre(*args):
  per_type = [_row_spec, _w_spec, _v_spec]
  return pl.pallas_call(
      _tc_pre_body,
      grid=(N // _TC_BLOCK,),
      in_specs=per_type + per_type,
      out_specs=[_row_spec, _row_spec],
      out_shape=_row_out,
  )(*args)


def _tc_call(*args):
  per_type = [_sum_spec, _cnt_spec, _row_spec, _w_spec, _v_spec, _v_spec]
  return pl.pallas_call(
      _tc_body,
      grid=(N // _TC_BLOCK,),
      in_specs=per_type + per_type,
      out_specs=[_row_spec, _row_spec],
      out_shape=_row_out,
  )(*args)


def kernel(x_user, x_item, edge_ui, edge_iu, Wl_ui, bl_ui, Wr_ui,
           Wl_iu, bl_iu, Wr_iu, g_user, b_user, g_item, b_item):
  zf = jnp.zeros((N, D), jnp.bfloat16)
  zc = jnp.zeros((N, CW), jnp.bfloat16)
  sum_i, cnt_i, sum_u, cnt_u = _sc_segment_sums(
      x_user.astype(jnp.bfloat16), x_item.astype(jnp.bfloat16),
      edge_ui, edge_iu, zf, zc)
  pre_u, pre_i = _tc_pre(x_user, Wr_iu, bl_iu, x_item, Wr_ui, bl_ui)
  out_u, out_i = _tc_call(
      sum_u, cnt_u, pre_u, Wl_iu, g_user, b_user,
      sum_i, cnt_i, pre_i, Wl_ui, g_item, b_item)
  return (out_u, out_i)


# speedup vs baseline: 1.7809x; 1.0858x over previous
import functools

import jax
import jax.numpy as jnp
from jax import lax
from jax.experimental import pallas as pl
from jax.experimental.pallas import tpu as pltpu
from jax.experimental.pallas import tpu_sc as plsc

N = 10000
D = 128
E = 320000
NS = 16
EPT = E // NS
K = 80
STEPS = EPT // K
NB = 5
CW = 32


def _sc_body(xu, xi, eui, eiu,
             sum_i, cnt_i, sum_u, cnt_u,
             acc, cacc, src_v, dst_v, rows, ones_v, semG, semS, semC):
  c = lax.axis_index("c")
  s = lax.axis_index("s")

  @pl.loop(0, K)
  def _(j):
    for l in range(4):
      rows[0, j, pl.ds(l * 32, 32)] = jnp.zeros((32,), jnp.bfloat16)
    ones_v[j, :] = jnp.zeros((32,), jnp.bfloat16)

  @pl.when(s < NS - 1)
  def _():
    @pl.loop(0, 8)
    def _(r):
      o = s * 640 + r * K
      pltpu.sync_copy(rows.at[0], acc.at[pl.ds(o, K)])
      pltpu.sync_copy(ones_v, cacc.at[pl.ds(o, K)])

  @pl.when(s == NS - 1)
  def _():
    @pl.loop(0, 5)
    def _(r):
      o = 9600 + r * K
      pltpu.sync_copy(rows.at[0], acc.at[pl.ds(o, K)])
      pltpu.sync_copy(ones_v, cacc.at[pl.ds(o, K)])

  @pl.loop(0, K)
  def _(j):
    ones_v[j, :] = jnp.ones((32,), jnp.bfloat16)

  plsc.subcore_barrier()

  def run_type(e_hbm, x_src):
    pltpu.sync_copy(e_hbm.at[0, pl.ds(s * EPT, EPT)], src_v)
    pltpu.sync_copy(e_hbm.at[1, pl.ds(s * EPT, EPT)], dst_v)

    def sidx(jq):
      return src_v.at[pl.ds(jq * K, K)]

    def didx(jq):
      return dst_v.at[pl.ds(jq * K, K)]

    for q in range(NB - 1):
      pltpu.async_copy(x_src.at[sidx(q)], rows.at[q], semG[q])

    @pl.loop(0, STEPS // NB)
    def _(t):
      j0 = NB * t
      for q in range(NB):
        jq = j0 + q
        qn = (q + NB - 1) % NB

        @pl.when(jq >= 1)
        def _():
          pltpu.make_async_copy(rows.at[qn], acc.at[didx(0)], semS[qn]).wait()
          pltpu.make_async_copy(ones_v, cacc.at[didx(0)], semC[qn]).wait()

        @pl.when(jq + NB - 1 < STEPS)
        def _():
          pltpu.async_copy(x_src.at[sidx(jq + NB - 1)], rows.at[qn], semG[qn])

        pltpu.make_async_copy(x_src.at[sidx(jq)], rows.at[q], semG[q]).wait()
        pltpu.async_copy(rows.at[q], acc.at[didx(jq)], semS[q], add=True)
        pltpu.async_copy(ones_v, cacc.at[didx(jq)], semC[q], add=True)

    pltpu.make_async_copy(rows.at[NB - 1], acc.at[didx(0)],
                          semS[NB - 1]).wait()
    pltpu.make_async_copy(ones_v, cacc.at[didx(0)], semC[NB - 1]).wait()

  @pl.when(c == 0)
  def _():
    run_type(eui, xu)

  @pl.when(c == 1)
  def _():
    run_type(eiu, xi)

  plsc.subcore_barrier()

  def write_out(sum_o, cnt_o):
    @pl.when(s < NS - 1)
    def _():
      pltpu.sync_copy(acc.at[pl.ds(s * 640, 640)], sum_o.at[pl.ds(s * 640, 640)])
      pltpu.sync_copy(cacc.at[pl.ds(s * 640, 640)], cnt_o.at[pl.ds(s * 640, 640)])

    @pl.when(s == NS - 1)
    def _():
      pltpu.sync_copy(acc.at[pl.ds(9600, 400)], sum_o.at[pl.ds(9600, 400)])
      pltpu.sync_copy(cacc.at[pl.ds(9600, 400)], cnt_o.at[pl.ds(9600, 400)])

  @pl.when(c == 0)
  def _():
    write_out(sum_i, cnt_i)

  @pl.when(c == 1)
  def _():
    write_out(sum_u, cnt_u)


_sc_segment_sums = pl.kernel(
    _sc_body,
    out_type=[
        jax.ShapeDtypeStruct((N, D), jnp.bfloat16),
        jax.ShapeDtypeStruct((N, CW), jnp.bfloat16),
        jax.ShapeDtypeStruct((N, D), jnp.bfloat16),
        jax.ShapeDtypeStruct((N, CW), jnp.bfloat16),
    ],
    mesh=plsc.VectorSubcoreMesh(core_axis_name="c", subcore_axis_name="s"),
    scratch_types=[
        pltpu.VMEM_SHARED((N, D), jnp.bfloat16),
        pltpu.VMEM_SHARED((N, CW), jnp.bfloat16),
        pltpu.VMEM((EPT,), jnp.int32),
        pltpu.VMEM((EPT,), jnp.int32),
        pltpu.VMEM((NB, K, D), jnp.bfloat16),
        pltpu.VMEM((K, CW), jnp.bfloat16),
        [pltpu.SemaphoreType.DMA] * NB,
        [pltpu.SemaphoreType.DMA] * NB,
        [pltpu.SemaphoreType.DMA] * NB,
    ],
    compiler_params=pltpu.CompilerParams(use_tc_tiling_on_sc=False),
)


def _matmul_t(a, w):
  return lax.dot_general(a, w, (((1,), (1,)), ((), ())),
                         preferred_element_type=jnp.float32,
                         precision=lax.Precision.HIGHEST)


def _tc_pre_body(xu, wr_iu, bl_iu, xi, wr_ui, bl_ui, pre_u, pre_i):
  pre_u[...] = _matmul_t(xu[...], wr_iu[...]) + bl_iu[...]
  pre_i[...] = _matmul_t(xi[...], wr_ui[...]) + bl_ui[...]


def _tc_body(sum_u, cnt_u, pre_u, wl_iu, g_u, b_u,
             sum_i, cnt_i, pre_i, wl_ui, g_i, b_i,
             out_u, out_i):
  def post(summed, cnt, pre, wl, g, b):
    mean = (summed.astype(jnp.float32)
            / jnp.maximum(cnt[:, 0:1].astype(jnp.float32), 1.0))
    y = _matmul_t(mean, wl) + pre
    mu = jnp.mean(y, axis=-1, keepdims=True)
    var = jnp.mean((y - mu) ** 2, axis=-1, keepdims=True)
    yn = (y - mu) * lax.rsqrt(var + 1e-5) * g + b
    return yn * 0.5 * (1.0 + lax.erf(yn * 0.7071067811865476))

  out_u[...] = post(sum_u[...], cnt_u[...], pre_u[...],
                    wl_iu[...], g_u[...], b_u[...])
  out_i[...] = post(sum_i[...], cnt_i[...], pre_i[...],
                    wl_ui[...], g_i[...], b_i[...])


_TC_BLOCK = 2000
_row_spec = pl.BlockSpec((_TC_BLOCK, D), lambda i: (i, 0))
_sum_spec = pl.BlockSpec((_TC_BLOCK, D), lambda i: (i, 0))
_cnt_spec = pl.BlockSpec((_TC_BLOCK, CW), lambda i: (i, 0))
_w_spec = pl.BlockSpec((D, D), lambda i: (0, 0))
_v_spec = pl.BlockSpec((D,), lambda i: (0,))
_row_out = [jax.ShapeDtypeStruct((N, D), jnp.float32),
            jax.ShapeDtypeStruct((N, D), jnp.float32)]


def _tc_pre(*args):
  per_type = [_row_spec, _w_spec, _v_spec]
  return pl.pallas_call(
      _tc_pre_body,
      grid=(N // _TC_BLOCK,),
      in_specs=per_type + per_type,
      out_specs=[_row_spec, _row_spec],
      out_shape=_row_out,
  )(*args)


def _tc_call(*args):
  per_type = [_sum_spec, _cnt_spec, _row_spec, _w_spec, _v_spec, _v_spec]
  return pl.pallas_call(
      _tc_body,
      grid=(N // _TC_BLOCK,),
      in_specs=per_type + per_type,
      out_specs=[_row_spec, _row_spec],
      out_shape=_row_out,
  )(*args)


def kernel(x_user, x_item, edge_ui, edge_iu, Wl_ui, bl_ui, Wr_ui,
           Wl_iu, bl_iu, Wr_iu, g_user, b_user, g_item, b_item):
  sum_i, cnt_i, sum_u, cnt_u = _sc_segment_sums(
      x_user.astype(jnp.bfloat16), x_item.astype(jnp.bfloat16),
      edge_ui, edge_iu)
  pre_u, pre_i = _tc_pre(x_user, Wr_iu, bl_iu, x_item, Wr_ui, bl_ui)
  out_u, out_i = _tc_call(
      sum_u, cnt_u, pre_u, Wl_iu, g_user, b_user,
      sum_i, cnt_i, pre_i, Wl_ui, g_item, b_item)
  return (out_u, out_i)
